# baseline (speedup 1.0000x reference)
import functools

import jax
import jax.numpy as jnp
from jax import lax
from jax.experimental import pallas as pl
from jax.experimental.pallas import tpu as pltpu
from jax.experimental.pallas import tpu_sc as plsc

N = 10000
NP = 10240
E = 320000
D_IN = 128
H1 = 8
C1 = 8
F1 = H1 * C1
G1 = F1 + H1
F2 = 16
G2 = F2 + H1

NC = 2
NS = 16
NW = NC * NS
EPW = E // NW
EB = 100
NB = EPW // EB
NPS = NP // NS
NST = NPS // 2

_f32 = jnp.float32
_i32 = jnp.int32


def _tc_a_body(x_ref, w1_ref, asbd_ref, adbd_ref, h1a_ref, a1d_ref):
    h = jnp.dot(x_ref[...], w1_ref[...], preferred_element_type=_f32)
    a1s = jnp.dot(h, asbd_ref[...], preferred_element_type=_f32)
    h1a_ref[...] = jnp.concatenate([h, a1s], axis=1)
    a1d_ref[...] = jnp.dot(h, adbd_ref[...], preferred_element_type=_f32)


def _tc_a(x, w1, as_bd, ad_bd):
    bn = 1280
    grid = NP // bn
    return pl.pallas_call(
        _tc_a_body,
        grid=(grid,),
        in_specs=[
            pl.BlockSpec((bn, D_IN), lambda i: (i, 0)),
            pl.BlockSpec((D_IN, F1), lambda i: (0, 0)),
            pl.BlockSpec((F1, H1), lambda i: (0, 0)),
            pl.BlockSpec((F1, H1), lambda i: (0, 0)),
        ],
        out_specs=[
            pl.BlockSpec((bn, G1), lambda i: (i, 0)),
            pl.BlockSpec((bn, H1), lambda i: (i, 0)),
        ],
        out_shape=[
            jax.ShapeDtypeStruct((NP, G1), _f32),
            jax.ShapeDtypeStruct((NP, H1), _f32),
        ],
    )(x, w1, as_bd, ad_bd)


def _sc_body(FW, GW,
             src_hbm, dst_hbm, tab_hbm, ad_hbm, z_hbm, accp_hbm,
             sp_acc, ev_src, ev_dst, g0, g1, gd0, gd1, m0, m1,
             stage, sg0, sg1):
    c = lax.axis_index("c")
    s = lax.axis_index("s")
    wid = c * NS + s
    row0 = s * NPS

    pltpu.sync_copy(z_hbm, stage)
    pltpu.sync_copy(stage, sp_acc.at[pl.ds(row0, NST)])
    pltpu.sync_copy(stage, sp_acc.at[pl.ds(row0 + NST, NST)])
    pltpu.sync_copy(src_hbm.at[wid], ev_src)
    pltpu.sync_copy(dst_hbm.at[wid], ev_dst)
    plsc.subcore_barrier()

    iota = jnp.arange(16, dtype=_i32)
    rows_lo = iota // 8
    cols8 = iota % 8

    gbuf = (g0, g1)
    gdbuf = (gd0, gd1)
    mbuf = (m0, m1)
    sems = (sg0, sg1)

    def issue(b, u):
        pltpu.async_copy(tab_hbm.at[ev_src.at[b]], gbuf[u], sems[u])
        pltpu.async_copy(ad_hbm.at[ev_dst.at[b]], gdbuf[u], sems[u])

    def drain(b, u):
        pltpu.make_async_copy(tab_hbm.at[ev_src.at[b]], gbuf[u], sems[u]).wait()
        pltpu.make_async_copy(ad_hbm.at[ev_dst.at[b]], gdbuf[u], sems[u]).wait()

    issue(0, 0)
    issue(1, 1)

    @pl.loop(0, NB // 2)
    def _blk(t):
        for u in range(2):
            b = 2 * t + u
            g, gd, m = gbuf[u], gdbuf[u], mbuf[u]
            drain(b, u)
            for j in range(EB * H1 // 16):
                r = rows_lo + (2 * j)
                av = plsc.load_gather(g, [r, cols8 + FW])
                bv = plsc.load_gather(gd, [r, cols8])
                ev = av + bv
                wv = jnp.exp(jnp.maximum(ev, 0.2 * ev))
                plsc.store_scatter(m, [r, cols8 + FW], wv)
            @pl.loop(0, EB, unroll=10)
            def _edge(e):
                erow = jnp.zeros((16,), _i32) + e
                for q in range(FW // 16):
                    cw = (rows_lo + 2 * q) if FW == F1 else cols8
                    wq = plsc.load_gather(m, [erow, cw + FW])
                    hq = g[e, pl.ds(16 * q, 16)]
                    m[e, pl.ds(16 * q, 16)] = wq * hq
            pltpu.sync_copy(m, sp_acc.at[ev_dst.at[b]], add=True)
            nxt = b + 2

            @pl.when(nxt < NB)
            def _():
                issue(nxt, u)

    plsc.subcore_barrier()
    pltpu.sync_copy(sp_acc.at[pl.ds(row0, NST)], stage)
    pltpu.sync_copy(stage, accp_hbm.at[c, pl.ds(row0, NST)])
    pltpu.sync_copy(sp_acc.at[pl.ds(row0 + NST, NST)], stage)
    pltpu.sync_copy(stage, accp_hbm.at[c, pl.ds(row0 + NST, NST)])


def _sc_edge(FW, GW, srcr, dstr, tab, ad, z):
    mesh = plsc.VectorSubcoreMesh(core_axis_name="c", subcore_axis_name="s")
    fn = functools.partial(
        pl.kernel,
        out_type=jax.ShapeDtypeStruct((NC, NP, GW), _f32),
        mesh=mesh,
        compiler_params=pltpu.CompilerParams(
            use_tc_tiling_on_sc=False, needs_layout_passes=False),
        scratch_types=[
            pltpu.VMEM_SHARED((NP, GW), _f32),
            pltpu.VMEM((NB, EB), _i32),
            pltpu.VMEM((NB, EB), _i32),
            pltpu.VMEM((EB, GW), _f32),
            pltpu.VMEM((EB, GW), _f32),
            pltpu.VMEM((EB, H1), _f32),
            pltpu.VMEM((EB, H1), _f32),
            pltpu.VMEM((EB, GW), _f32),
            pltpu.VMEM((EB, GW), _f32),
            pltpu.VMEM((NST, GW), _f32),
            pltpu.SemaphoreType.DMA,
            pltpu.SemaphoreType.DMA,
        ],
    )(functools.partial(_sc_body, FW, GW))
    return fn(srcr, dstr, tab, ad, z)


def _tc_b_body(accp_ref, b1_ref, w2_ref, as2_ref, ad2_ref, r8_ref,
               h2a_ref, a2d_ref):
    fused = accp_ref[0] + accp_ref[1]
    acc = fused[:, 0:F1]
    den = fused[:, F1:G1]
    dinv = 1.0 / (den + 1e-16)
    drep = jnp.dot(dinv, r8_ref[...], preferred_element_type=_f32)
    xact = acc * drep + b1_ref[...]
    act = jnp.where(xact > 0, xact, jnp.exp(xact) - 1.0)
    h2 = jnp.dot(act, w2_ref[...], preferred_element_type=_f32)
    a2s = jnp.dot(h2, as2_ref[...], preferred_element_type=_f32)
    a2d = jnp.dot(h2, ad2_ref[...], preferred_element_type=_f32)
    a2s8 = jnp.concatenate([a2s] * H1, axis=1)
    h2a_ref[...] = jnp.concatenate([h2, a2s8], axis=1)
    a2d_ref[...] = jnp.concatenate([a2d] * H1, axis=1)


def _tc_b(accp, b1, w2, as2, ad2, r8):
    bn = 1280
    grid = NP // bn
    return pl.pallas_call(
        _tc_b_body,
        grid=(grid,),
        in_specs=[
            pl.BlockSpec((NC, bn, G1), lambda i: (0, i, 0)),
            pl.BlockSpec((1, F1), lambda i: (0, 0)),
            pl.BlockSpec((F1, F2), lambda i: (0, 0)),
            pl.BlockSpec((F2, 1), lambda i: (0, 0)),
            pl.BlockSpec((F2, 1), lambda i: (0, 0)),
            pl.BlockSpec((H1, F1), lambda i: (0, 0)),
        ],
        out_specs=[
            pl.BlockSpec((bn, G2), lambda i: (i, 0)),
            pl.BlockSpec((bn, H1), lambda i: (i, 0)),
        ],
        out_shape=[
            jax.ShapeDtypeStruct((NP, G2), _f32),
            jax.ShapeDtypeStruct((NP, H1), _f32),
        ],
    )(accp, b1, w2, as2, ad2, r8)


def _tc_c_body(accp_ref, b2_ref, out_ref):
    fused = accp_ref[0] + accp_ref[1]
    acc = fused[:, 0:F2]
    den = fused[:, F2:F2 + 1]
    logits = acc * (1.0 / (den + 1e-16)) + b2_ref[...]
    m = jnp.max(logits, axis=1, keepdims=True)
    sh = logits - m
    out_ref[...] = sh - jnp.log(jnp.sum(jnp.exp(sh), axis=1, keepdims=True))


def _tc_c(accp2, b2):
    return pl.pallas_call(
        _tc_c_body,
        grid=(1,),
        in_specs=[
            pl.BlockSpec((NC, NP, G2), lambda i: (0, 0, 0)),
            pl.BlockSpec((1, F2), lambda i: (0, 0)),
        ],
        out_specs=pl.BlockSpec((NP, F2), lambda i: (0, 0)),
        out_shape=jax.ShapeDtypeStruct((NP, F2), _f32),
    )(accp2, b2)


def kernel(x, edge_index, W1, att_src1, att_dst1, b1, W2, att_src2, att_dst2, b2):
    r8 = jnp.repeat(jnp.eye(H1, dtype=_f32), C1, axis=1)
    as_bd = r8.T * att_src1.reshape(-1)[:, None]
    ad_bd = r8.T * att_dst1.reshape(-1)[:, None]
    as2 = att_src2.reshape(F2, 1)
    ad2 = att_dst2.reshape(F2, 1)
    er = edge_index.reshape(2, NW, NB, EB)
    srcr, dstr = er[0], er[1]
    z72 = jnp.zeros((NST, G1), _f32)
    z24 = jnp.zeros((NST, G2), _f32)

    xp = jnp.concatenate([x, jnp.zeros((NP - N, D_IN), _f32)], axis=0)
    h1a, a1d = _tc_a(xp, W1, as_bd, ad_bd)
    accp = _sc_edge(F1, G1, srcr, dstr, h1a, a1d, z72)
    h2a, a2d8 = _tc_b(accp, b1.reshape(1, F1), W2, as2, ad2, r8)
    accp2 = _sc_edge(F2, G2, srcr, dstr, h2a, a2d8, z24)
    out = _tc_c(accp2, b2.reshape(1, F2))
    return out[:N]


# speedup vs baseline: 1.0851x; 1.0851x over previous
import functools

import jax
import jax.numpy as jnp
from jax import lax
from jax.experimental import pallas as pl
from jax.experimental.pallas import tpu as pltpu
from jax.experimental.pallas import tpu_sc as plsc

N = 10000
NP = 10240
E = 320000
D_IN = 128
H1 = 8
C1 = 8
F1 = H1 * C1
G1 = F1 + H1
F2 = 16
G2 = F2 + H1

NC = 2
NS = 16
NW = NC * NS
EPW = E // NW
EB = 100
NB = EPW // EB
NPS = NP // NS
NST = NPS // 2

_f32 = jnp.float32
_i32 = jnp.int32


def _tc_a_body(x_ref, w1_ref, asbd_ref, adbd_ref, h1a_ref, a1d_ref):
    h = jnp.dot(x_ref[...], w1_ref[...], preferred_element_type=_f32)
    a1s = jnp.dot(h, asbd_ref[...], preferred_element_type=_f32)
    h1a_ref[...] = jnp.concatenate([h, a1s], axis=1)
    a1d_ref[...] = jnp.dot(h, adbd_ref[...], preferred_element_type=_f32)


def _tc_a(x, w1, as_bd, ad_bd):
    bn = 1280
    grid = NP // bn
    return pl.pallas_call(
        _tc_a_body,
        grid=(grid,),
        in_specs=[
            pl.BlockSpec((bn, D_IN), lambda i: (i, 0)),
            pl.BlockSpec((D_IN, F1), lambda i: (0, 0)),
            pl.BlockSpec((F1, H1), lambda i: (0, 0)),
            pl.BlockSpec((F1, H1), lambda i: (0, 0)),
        ],
        out_specs=[
            pl.BlockSpec((bn, G1), lambda i: (i, 0)),
            pl.BlockSpec((bn, H1), lambda i: (i, 0)),
        ],
        out_shape=[
            jax.ShapeDtypeStruct((NP, G1), _f32),
            jax.ShapeDtypeStruct((NP, H1), _f32),
        ],
    )(x, w1, as_bd, ad_bd)


def _sc_body(FW, GW,
             src_hbm, dst_hbm, tab_hbm, ad_hbm, z_hbm, accp_hbm,
             sp_acc, ev_src, ev_dst, g0, g1, gd0, gd1, m0, m1,
             stage, sg0, sg1, ss0, ss1):
    c = lax.axis_index("c")
    s = lax.axis_index("s")
    wid = c * NS + s
    row0 = s * NPS

    pltpu.sync_copy(z_hbm, stage)
    pltpu.sync_copy(stage, sp_acc.at[pl.ds(row0, NST)])
    pltpu.sync_copy(stage, sp_acc.at[pl.ds(row0 + NST, NST)])
    pltpu.sync_copy(src_hbm.at[wid], ev_src)
    pltpu.sync_copy(dst_hbm.at[wid], ev_dst)
    plsc.subcore_barrier()

    iota = jnp.arange(16, dtype=_i32)
    rows_lo = iota // 8
    cols8 = iota % 8

    gbuf = (g0, g1)
    gdbuf = (gd0, gd1)
    mbuf = (m0, m1)
    sems = (sg0, sg1)
    ssems = (ss0, ss1)

    def issue(b, u):
        pltpu.async_copy(tab_hbm.at[ev_src.at[b]], gbuf[u], sems[u])
        pltpu.async_copy(ad_hbm.at[ev_dst.at[b]], gdbuf[u], sems[u])

    def drain(b, u):
        pltpu.make_async_copy(tab_hbm.at[ev_src.at[b]], gbuf[u], sems[u]).wait()
        pltpu.make_async_copy(ad_hbm.at[ev_dst.at[b]], gdbuf[u], sems[u]).wait()

    issue(0, 0)
    issue(1, 1)

    @pl.loop(0, NB // 2)
    def _blk(t):
        for u in range(2):
            b = 2 * t + u
            g, gd, m = gbuf[u], gdbuf[u], mbuf[u]
            drain(b, u)

            @pl.when(b >= 2)
            def _():
                pltpu.make_async_copy(m, sp_acc.at[ev_dst.at[b]],
                                      ssems[u]).wait()
            for j in range(EB * H1 // 16):
                r = rows_lo + (2 * j)
                av = plsc.load_gather(g, [r, cols8 + FW])
                bv = plsc.load_gather(gd, [r, cols8])
                ev = av + bv
                wv = jnp.exp(jnp.maximum(ev, 0.2 * ev))
                plsc.store_scatter(m, [r, cols8 + FW], wv)
            @pl.loop(0, EB)
            def _edge(e):
                erow = jnp.zeros((16,), _i32) + e
                for q in range(FW // 16):
                    cw = (rows_lo + 2 * q) if FW == F1 else cols8
                    wq = plsc.load_gather(m, [erow, cw + FW])
                    hq = g[e, pl.ds(16 * q, 16)]
                    m[e, pl.ds(16 * q, 16)] = wq * hq
            pltpu.async_copy(m, sp_acc.at[ev_dst.at[b]], ssems[u], add=True)
            nxt = b + 2

            @pl.when(nxt < NB)
            def _():
                issue(nxt, u)

    for u in range(2):
        pltpu.make_async_copy(mbuf[u], sp_acc.at[ev_dst.at[0]],
                              ssems[u]).wait()
    plsc.subcore_barrier()
    pltpu.sync_copy(sp_acc.at[pl.ds(row0, NST)], stage)
    pltpu.sync_copy(stage, accp_hbm.at[c, pl.ds(row0, NST)])
    pltpu.sync_copy(sp_acc.at[pl.ds(row0 + NST, NST)], stage)
    pltpu.sync_copy(stage, accp_hbm.at[c, pl.ds(row0 + NST, NST)])


def _sc_edge(FW, GW, srcr, dstr, tab, ad, z):
    mesh = plsc.VectorSubcoreMesh(core_axis_name="c", subcore_axis_name="s")
    fn = functools.partial(
        pl.kernel,
        out_type=jax.ShapeDtypeStruct((NC, NP, GW), _f32),
        mesh=mesh,
        compiler_params=pltpu.CompilerParams(
            use_tc_tiling_on_sc=False, needs_layout_passes=False),
        scratch_types=[
            pltpu.VMEM_SHARED((NP, GW), _f32),
            pltpu.VMEM((NB, EB), _i32),
            pltpu.VMEM((NB, EB), _i32),
            pltpu.VMEM((EB, GW), _f32),
            pltpu.VMEM((EB, GW), _f32),
            pltpu.VMEM((EB, H1), _f32),
            pltpu.VMEM((EB, H1), _f32),
            pltpu.VMEM((EB, GW), _f32),
            pltpu.VMEM((EB, GW), _f32),
            pltpu.VMEM((NST, GW), _f32),
            pltpu.SemaphoreType.DMA,
            pltpu.SemaphoreType.DMA,
            pltpu.SemaphoreType.DMA,
            pltpu.SemaphoreType.DMA,
        ],
    )(functools.partial(_sc_body, FW, GW))
    return fn(srcr, dstr, tab, ad, z)


def _tc_b_body(accp_ref, b1_ref, w2_ref, as2_ref, ad2_ref, r8_ref,
               h2a_ref, a2d_ref):
    fused = accp_ref[0] + accp_ref[1]
    acc = fused[:, 0:F1]
    den = fused[:, F1:G1]
    dinv = 1.0 / (den + 1e-16)
    drep = jnp.dot(dinv, r8_ref[...], preferred_element_type=_f32)
    xact = acc * drep + b1_ref[...]
    act = jnp.where(xact > 0, xact, jnp.exp(xact) - 1.0)
    h2 = jnp.dot(act, w2_ref[...], preferred_element_type=_f32)
    a2s = jnp.dot(h2, as2_ref[...], preferred_element_type=_f32)
    a2d = jnp.dot(h2, ad2_ref[...], preferred_element_type=_f32)
    a2s8 = jnp.concatenate([a2s] * H1, axis=1)
    h2a_ref[...] = jnp.concatenate([h2, a2s8], axis=1)
    a2d_ref[...] = jnp.concatenate([a2d] * H1, axis=1)


def _tc_b(accp, b1, w2, as2, ad2, r8):
    bn = 1280
    grid = NP // bn
    return pl.pallas_call(
        _tc_b_body,
        grid=(grid,),
        in_specs=[
            pl.BlockSpec((NC, bn, G1), lambda i: (0, i, 0)),
            pl.BlockSpec((1, F1), lambda i: (0, 0)),
            pl.BlockSpec((F1, F2), lambda i: (0, 0)),
            pl.BlockSpec((F2, 1), lambda i: (0, 0)),
            pl.BlockSpec((F2, 1), lambda i: (0, 0)),
            pl.BlockSpec((H1, F1), lambda i: (0, 0)),
        ],
        out_specs=[
            pl.BlockSpec((bn, G2), lambda i: (i, 0)),
            pl.BlockSpec((bn, H1), lambda i: (i, 0)),
        ],
        out_shape=[
            jax.ShapeDtypeStruct((NP, G2), _f32),
            jax.ShapeDtypeStruct((NP, H1), _f32),
        ],
    )(accp, b1, w2, as2, ad2, r8)


def _tc_c_body(accp_ref, b2_ref, out_ref):
    fused = accp_ref[0] + accp_ref[1]
    acc = fused[:, 0:F2]
    den = fused[:, F2:F2 + 1]
    logits = acc * (1.0 / (den + 1e-16)) + b2_ref[...]
    m = jnp.max(logits, axis=1, keepdims=True)
    sh = logits - m
    out_ref[...] = sh - jnp.log(jnp.sum(jnp.exp(sh), axis=1, keepdims=True))


def _tc_c(accp2, b2):
    return pl.pallas_call(
        _tc_c_body,
        grid=(1,),
        in_specs=[
            pl.BlockSpec((NC, NP, G2), lambda i: (0, 0, 0)),
            pl.BlockSpec((1, F2), lambda i: (0, 0)),
        ],
        out_specs=pl.BlockSpec((NP, F2), lambda i: (0, 0)),
        out_shape=jax.ShapeDtypeStruct((NP, F2), _f32),
    )(accp2, b2)


def kernel(x, edge_index, W1, att_src1, att_dst1, b1, W2, att_src2, att_dst2, b2):
    r8 = jnp.repeat(jnp.eye(H1, dtype=_f32), C1, axis=1)
    as_bd = r8.T * att_src1.reshape(-1)[:, None]
    ad_bd = r8.T * att_dst1.reshape(-1)[:, None]
    as2 = att_src2.reshape(F2, 1)
    ad2 = att_dst2.reshape(F2, 1)
    er = edge_index.reshape(2, NW, NB, EB)
    srcr, dstr = er[0], er[1]
    z72 = jnp.zeros((NST, G1), _f32)
    z24 = jnp.zeros((NST, G2), _f32)

    xp = jnp.concatenate([x, jnp.zeros((NP - N, D_IN), _f32)], axis=0)
    h1a, a1d = _tc_a(xp, W1, as_bd, ad_bd)
    accp = _sc_edge(F1, G1, srcr, dstr, h1a, a1d, z72)
    h2a, a2d8 = _tc_b(accp, b1.reshape(1, F1), W2, as2, ad2, r8)
    accp2 = _sc_edge(F2, G2, srcr, dstr, h2a, a2d8, z24)
    out = _tc_c(accp2, b2.reshape(1, F2))
    return out[:N]
